# baseline (speedup 1.0000x reference)
import functools

import jax
import jax.numpy as jnp
from jax import lax
from jax.experimental import pallas as pl
from jax.experimental.pallas import tpu as pltpu
from jax.experimental.pallas import tpu_sc as plsc

N_NODES = 10000
MAX_DEG = 32
D_FEAT = 128
_INV = 1.0 / MAX_DEG

_TC_N = 8464
_BN = 368
_SC_N = N_NODES - _TC_N

_NC, _NS, _L = 2, 16, 16
_NW = _NC * _NS
_NPW = _SC_N // _NW
_TILE = 8
_T = _NPW // _TILE
_NBUF = 3


def _tc_body(x_ref, o_ref):
    o_ref[...] = jnp.sum(x_ref[...], axis=1) * _INV


_mesh = plsc.VectorSubcoreMesh(core_axis_name="c", subcore_axis_name="s")


@functools.partial(
    pl.kernel,
    mesh=_mesh,
    out_type=jax.ShapeDtypeStruct((_SC_N, D_FEAT), jnp.float32),
    scratch_types=[
        pltpu.VMEM((_NBUF, _TILE, MAX_DEG, D_FEAT), jnp.float32),
        pltpu.VMEM((_NBUF, _TILE, D_FEAT), jnp.float32),
        pltpu.SemaphoreType.DMA((_NBUF,)),
        pltpu.SemaphoreType.DMA((_NBUF,)),
    ],
)
def _sc_mean(mail_hbm, out_hbm, in_v, out_v, in_sem, out_sem):
    wid = lax.axis_index("s") * _NC + lax.axis_index("c")
    base = _TC_N + wid * _NPW
    obase = wid * _NPW

    def in_copy(slot, g):
        return pltpu.make_async_copy(
            mail_hbm.at[pl.ds(base + g * _TILE, _TILE)],
            in_v.at[slot], in_sem.at[slot])

    def out_copy(slot, g):
        return pltpu.make_async_copy(
            out_v.at[slot],
            out_hbm.at[pl.ds(obase + g * _TILE, _TILE)], out_sem.at[slot])

    for b in range(min(_NBUF, _T)):
        in_copy(b, b).start()

    for g in range(_T):
        slot = g % _NBUF
        in_copy(slot, g).wait()
        if g >= _NBUF:
            out_copy(slot, g - _NBUF).wait()

        @plsc.parallel_loop(0, _TILE, step=1)
        def _node(n):
            for c in range(D_FEAT // _L):
                acc = in_v[slot, n, 0, pl.ds(c * _L, _L)]
                for j in range(1, MAX_DEG):
                    acc = acc + in_v[slot, n, j, pl.ds(c * _L, _L)]
                out_v[slot, n, pl.ds(c * _L, _L)] = acc * _INV

        out_copy(slot, g).start()
        nxt = g + _NBUF
        if nxt < _T:
            in_copy(slot, nxt).start()

    for g in range(_T - _NBUF, _T):
        out_copy(g % _NBUF, g).wait()


def kernel(mailbox_m):
    tc_out = pl.pallas_call(
        _tc_body,
        grid=(_TC_N // _BN,),
        in_specs=[pl.BlockSpec((_BN, MAX_DEG, D_FEAT), lambda i: (i, 0, 0))],
        out_specs=pl.BlockSpec((_BN, D_FEAT), lambda i: (i, 0)),
        out_shape=jax.ShapeDtypeStruct((_TC_N, D_FEAT), jnp.float32),
    )(mailbox_m)
    sc_out = _sc_mean(mailbox_m)
    return jnp.concatenate([tc_out, sc_out], axis=0)


# speedup vs baseline: 1.0110x; 1.0110x over previous
import functools

import jax
import jax.numpy as jnp
from jax import lax
from jax.experimental import pallas as pl
from jax.experimental.pallas import tpu as pltpu
from jax.experimental.pallas import tpu_sc as plsc

N_NODES = 10000
MAX_DEG = 32
D_FEAT = 128
_INV = 1.0 / MAX_DEG

_NC, _NS, _L = 2, 16, 16
_NW = _NC * _NS
_TILE = 8

_SC_N = _NW * _TILE
_TC_N = N_NODES - _SC_N
_BN = 336


def _tc_body(x_ref, o_ref):
    o_ref[...] = jnp.sum(x_ref[...], axis=1) * _INV


_mesh = plsc.VectorSubcoreMesh(core_axis_name="c", subcore_axis_name="s")


@functools.partial(
    pl.kernel,
    mesh=_mesh,
    out_type=jax.ShapeDtypeStruct((_SC_N, D_FEAT), jnp.float32),
    scratch_types=[
        pltpu.VMEM((_TILE, MAX_DEG, D_FEAT), jnp.float32),
        pltpu.VMEM((_TILE, D_FEAT), jnp.float32),
    ],
)
def _sc_mean(mail_hbm, out_hbm, in_v, out_v):
    wid = lax.axis_index("s") * _NC + lax.axis_index("c")
    base = _TC_N + wid * _TILE
    obase = wid * _TILE

    pltpu.sync_copy(mail_hbm.at[pl.ds(base, _TILE)], in_v)

    @plsc.parallel_loop(0, _TILE, step=1)
    def _node(n):
        for c in range(D_FEAT // _L):
            acc = in_v[n, 0, pl.ds(c * _L, _L)]
            for j in range(1, MAX_DEG):
                acc = acc + in_v[n, j, pl.ds(c * _L, _L)]
            out_v[n, pl.ds(c * _L, _L)] = acc * _INV

    pltpu.sync_copy(out_v, out_hbm.at[pl.ds(obase, _TILE)])


def kernel(mailbox_m):
    tc_out = pl.pallas_call(
        _tc_body,
        grid=(_TC_N // _BN,),
        in_specs=[pl.BlockSpec((_BN, MAX_DEG, D_FEAT), lambda i: (i, 0, 0))],
        out_specs=pl.BlockSpec((_BN, D_FEAT), lambda i: (i, 0)),
        out_shape=jax.ShapeDtypeStruct((_TC_N, D_FEAT), jnp.float32),
    )(mailbox_m)
    sc_out = _sc_mean(mailbox_m)
    return jnp.concatenate([tc_out, sc_out], axis=0)


# speedup vs baseline: 1.0555x; 1.0440x over previous
import functools

import jax
import jax.numpy as jnp
from jax import lax
from jax.experimental import pallas as pl
from jax.experimental.pallas import tpu as pltpu
from jax.experimental.pallas import tpu_sc as plsc

N_NODES = 10000
MAX_DEG = 32
D_FEAT = 128
_INV = 1.0 / MAX_DEG

_NC, _NS, _L = 2, 16, 16
_NW = _NC * _NS
_TILE = 8

_SC_N = _NW * _TILE
_TC_N = N_NODES - _SC_N
_BN = 336


def _tc_body(x_ref, o_ref):
    o_ref[...] = jnp.sum(x_ref[...], axis=1) * _INV


_mesh = plsc.VectorSubcoreMesh(core_axis_name="c", subcore_axis_name="s")


@functools.partial(
    pl.kernel,
    mesh=_mesh,
    out_type=jax.ShapeDtypeStruct((_SC_N, D_FEAT), jnp.float32),
    scratch_types=[
        pltpu.VMEM((_TILE, MAX_DEG, D_FEAT), jnp.float32),
        pltpu.VMEM((_TILE, D_FEAT), jnp.float32),
    ],
)
def _sc_mean(mail_hbm, out_hbm, in_v, out_v):
    wid = lax.axis_index("s") * _NC + lax.axis_index("c")
    base = _TC_N + wid * _TILE
    obase = wid * _TILE

    pltpu.sync_copy(mail_hbm.at[pl.ds(base, _TILE)], in_v)

    @plsc.parallel_loop(0, _TILE, step=1)
    def _node(n):
        for c in range(D_FEAT // _L):
            acc = in_v[n, 0, pl.ds(c * _L, _L)]
            for j in range(1, MAX_DEG):
                acc = acc + in_v[n, j, pl.ds(c * _L, _L)]
            out_v[n, pl.ds(c * _L, _L)] = acc * _INV

    pltpu.sync_copy(out_v, out_hbm.at[pl.ds(obase, _TILE)])


def kernel(mailbox_m):
    tc_out = pl.pallas_call(
        _tc_body,
        grid=(_TC_N // _BN,),
        in_specs=[pl.BlockSpec((_BN, MAX_DEG, D_FEAT), lambda i: (i, 0, 0))],
        out_specs=pl.BlockSpec((_BN, D_FEAT), lambda i: (i, 0)),
        out_shape=jax.ShapeDtypeStruct((N_NODES, D_FEAT), jnp.float32),
    )(mailbox_m)
    sc_out = _sc_mean(mailbox_m)
    return lax.dynamic_update_slice(tc_out, sc_out, (_TC_N, 0))


# speedup vs baseline: 1.0652x; 1.0091x over previous
import functools

import jax
import jax.numpy as jnp
from jax import lax
from jax.experimental import pallas as pl
from jax.experimental.pallas import tpu as pltpu
from jax.experimental.pallas import tpu_sc as plsc

N_NODES = 10000
MAX_DEG = 32
D_FEAT = 128
_INV = 1.0 / MAX_DEG

_NC, _NS, _L = 2, 16, 16
_NW = _NC * _NS
_TILE = 8

_SC_N = _NW * _TILE
_TC_N = N_NODES - _SC_N
_BN = 464


def _tc_body(x_ref, o_ref):
    o_ref[...] = jnp.sum(x_ref[...], axis=1) * _INV


_mesh = plsc.VectorSubcoreMesh(core_axis_name="c", subcore_axis_name="s")


@functools.partial(
    pl.kernel,
    mesh=_mesh,
    out_type=jax.ShapeDtypeStruct((_SC_N, D_FEAT), jnp.float32),
    scratch_types=[
        pltpu.VMEM((_TILE, MAX_DEG, D_FEAT), jnp.float32),
        pltpu.VMEM((_TILE, D_FEAT), jnp.float32),
    ],
)
def _sc_mean(mail_hbm, out_hbm, in_v, out_v):
    wid = lax.axis_index("s") * _NC + lax.axis_index("c")
    base = _TC_N + wid * _TILE
    obase = wid * _TILE

    pltpu.sync_copy(mail_hbm.at[pl.ds(base, _TILE)], in_v)

    @plsc.parallel_loop(0, _TILE, step=1)
    def _node(n):
        for c in range(D_FEAT // _L):
            acc = in_v[n, 0, pl.ds(c * _L, _L)]
            for j in range(1, MAX_DEG):
                acc = acc + in_v[n, j, pl.ds(c * _L, _L)]
            out_v[n, pl.ds(c * _L, _L)] = acc * _INV

    pltpu.sync_copy(out_v, out_hbm.at[pl.ds(obase, _TILE)])


def kernel(mailbox_m):
    sc_out = _sc_mean(mailbox_m)
    tc_out = pl.pallas_call(
        _tc_body,
        grid=(_TC_N // _BN,),
        in_specs=[pl.BlockSpec((_BN, MAX_DEG, D_FEAT), lambda i: (i, 0, 0))],
        out_specs=pl.BlockSpec((_BN, D_FEAT), lambda i: (i, 0)),
        out_shape=jax.ShapeDtypeStruct((N_NODES, D_FEAT), jnp.float32),
    )(mailbox_m)
    return lax.dynamic_update_slice(tc_out, sc_out, (_TC_N, 0))
